# Initial kernel scaffold; baseline (speedup 1.0000x reference)
import jax
import jax.numpy as jnp
from jax.experimental import pallas as pl


def kernel(inp, label):
    raise NotImplementedError("write your pallas kernel here")


# speedup vs baseline: 4.0247x; 4.0247x over previous
import functools

import jax
import jax.numpy as jnp
from jax.experimental import pallas as pl
from jax.experimental.pallas import tpu as pltpu

_MARGIN = 0.25
_SCALE = 64.0


def _loss_kernel(label_ref, x_ref, out_ref, m_ref, s_ref, t_ref, *, n_cols,
                 block_cols):
    k = pl.program_id(0)
    nk = pl.num_programs(0)

    @pl.when(k == 0)
    def _init():
        m_ref[...] = jnp.full(m_ref.shape, -jnp.inf, m_ref.dtype)
        s_ref[...] = jnp.zeros(s_ref.shape, s_ref.dtype)
        t_ref[...] = jnp.zeros(t_ref.shape, t_ref.dtype)

    x = x_ref[...]
    col = jax.lax.broadcasted_iota(jnp.int32, x.shape, 1) + k * block_cols
    lab = label_ref[...]
    is_lab = col == lab
    valid = col < n_cols
    dense = jnp.maximum(x + _MARGIN, 0.0) * (x - _MARGIN) * _SCALE
    spec = jnp.maximum((1.0 + _MARGIN) - x, 0.0) * (x - (1.0 - _MARGIN)) * _SCALE
    logit = jnp.where(is_lab, spec, dense)
    logit = jnp.where(valid, logit, -jnp.inf)
    t_ref[...] += jnp.sum(jnp.where(is_lab, spec, 0.0), axis=1, keepdims=True)
    bm = jnp.max(logit, axis=1, keepdims=True)
    m_old = m_ref[...]
    m_new = jnp.maximum(m_old, bm)
    s_ref[...] = s_ref[...] * jnp.exp(m_old - m_new) + jnp.sum(
        jnp.exp(logit - m_new), axis=1, keepdims=True)
    m_ref[...] = m_new

    @pl.when(k == nk - 1)
    def _fin():
        loss = m_ref[...] + jnp.log(s_ref[...]) - t_ref[...]
        out_ref[0, 0] = jnp.sum(loss) / loss.shape[0]


def kernel(inp, label):
    b, c = inp.shape
    block_cols = 2048
    nk = pl.cdiv(c, block_cols)
    lab2 = label.astype(jnp.int32).reshape(b, 1)
    out = pl.pallas_call(
        functools.partial(_loss_kernel, n_cols=c, block_cols=block_cols),
        grid=(nk,),
        in_specs=[
            pl.BlockSpec((b, 1), lambda k: (0, 0)),
            pl.BlockSpec((b, block_cols), lambda k: (0, k)),
        ],
        out_specs=pl.BlockSpec(memory_space=pltpu.SMEM),
        out_shape=jax.ShapeDtypeStruct((1, 1), jnp.float32),
        scratch_shapes=[
            pltpu.VMEM((b, 1), jnp.float32),
            pltpu.VMEM((b, 1), jnp.float32),
            pltpu.VMEM((b, 1), jnp.float32),
        ],
    )(lab2, inp)
    return out[0, 0]


# speedup vs baseline: 4.6042x; 1.1440x over previous
import functools

import jax
import jax.numpy as jnp
from jax.experimental import pallas as pl
from jax.experimental.pallas import tpu as pltpu

_M = 0.25
_A = 64.0 * 1.4426950408889634
_LN2 = 0.6931471805599453
_NEG_INF = float("-inf")


def _loss_kernel(label_ref, x_ref, out_ref, m_ref, s_ref, g_ref, *, n_cols,
                 block_cols):
    k = pl.program_id(0)
    nk = pl.num_programs(0)

    @pl.when(k == 0)
    def _init():
        m_ref[...] = jnp.full(m_ref.shape, _NEG_INF, m_ref.dtype)
        s_ref[...] = jnp.zeros(s_ref.shape, s_ref.dtype)
        g_ref[...] = jnp.zeros(g_ref.shape, g_ref.dtype)

    def _accum(mask_invalid):
        x = x_ref[...]
        labloc = label_ref[...] - k * block_cols
        col = jax.lax.broadcasted_iota(jnp.int32, x.shape, 1)
        is_lab = col == labloc
        q = x * x * _A - (_A / 16.0)
        dense = jnp.where(x > -_M, q, 0.0)
        if mask_invalid:
            drop = is_lab | (col >= n_cols - k * block_cols)
        else:
            drop = is_lab
        l2 = jnp.where(drop, _NEG_INF, dense)
        g_ref[...] += jnp.sum(jnp.where(is_lab, x, 0.0), axis=1,
                              keepdims=True)
        bm = jnp.max(l2, axis=1, keepdims=True)
        m_old = m_ref[...]
        m_new = jnp.maximum(m_old, bm)
        s_ref[...] = s_ref[...] * jnp.exp2(m_old - m_new) + jnp.sum(
            jnp.exp2(l2 - m_new), axis=1, keepdims=True)
        m_ref[...] = m_new

    @pl.when(k < nk - 1)
    def _main():
        _accum(False)

    @pl.when(k == nk - 1)
    def _last():
        _accum(True)

        g = g_ref[...]
        spec2 = jnp.where(g < 1.0 + _M,
                          (2.0 * _A) * g - g * g * _A - 0.9375 * _A, 0.0)
        m2 = m_ref[...]
        s_true = s_ref[...] + jnp.exp2(spec2 - m2)
        loss = (m2 + jnp.log2(s_true) - spec2) * _LN2
        out_ref[0, 0] = jnp.sum(loss) / loss.shape[0]


def kernel(inp, label):
    b, c = inp.shape
    block_cols = 2048
    nk = pl.cdiv(c, block_cols)
    lab2 = label.astype(jnp.int32).reshape(b, 1)
    out = pl.pallas_call(
        functools.partial(_loss_kernel, n_cols=c, block_cols=block_cols),
        grid=(nk,),
        in_specs=[
            pl.BlockSpec((b, 1), lambda k: (0, 0)),
            pl.BlockSpec((b, block_cols), lambda k: (0, k)),
        ],
        out_specs=pl.BlockSpec(memory_space=pltpu.SMEM),
        out_shape=jax.ShapeDtypeStruct((1, 1), jnp.float32),
        scratch_shapes=[
            pltpu.VMEM((b, 1), jnp.float32),
            pltpu.VMEM((b, 1), jnp.float32),
            pltpu.VMEM((b, 1), jnp.float32),
        ],
    )(lab2, inp)
    return out[0, 0]
